# baseline (speedup 1.0000x reference)
import functools
import math

import jax
import jax.numpy as jnp
from jax import lax
from jax.experimental import pallas as pl
from jax.experimental.pallas import tpu as pltpu
from jax.experimental.pallas import tpu_sc as plsc

LANES = 16
NBUF = 3


@functools.lru_cache(maxsize=None)
def _make_sc_kernel(batch: int, seq_len: int, d_model: int):
    info = plsc.get_sparse_core_info()
    num_workers = info.num_cores * info.num_subcores
    l_per_w = seq_len // num_workers
    half = l_per_w // 2
    n_chunks = batch * 2
    n_slices = d_model // LANES
    scale = math.sqrt(float(d_model))
    mesh = plsc.VectorSubcoreMesh(core_axis_name="c", subcore_axis_name="s")

    @functools.partial(
        pl.kernel,
        mesh=mesh,
        out_type=jax.ShapeDtypeStruct((batch * seq_len, d_model), jnp.float32),
        scratch_types=[
            pltpu.VMEM((batch, l_per_w), jnp.int32),
            pltpu.VMEM((l_per_w, d_model), jnp.float32),
        ]
        + [pltpu.VMEM((half, d_model), jnp.float32) for _ in range(NBUF)]
        + [pltpu.SemaphoreType.DMA for _ in range(2 + 2 * NBUF)],
    )
    def k(src_hbm, table_hbm, pe_hbm, out_hbm, idx_v, pe_v, r0, r1, r2,
          isem, psem, g0, g1, g2, s0, s1, s2):
        rows = [r0, r1, r2]
        gsem = [g0, g1, g2]
        ssem = [s0, s1, s2]
        wid = lax.axis_index("s") * info.num_cores + lax.axis_index("c")
        lw = wid * l_per_w

        pe_desc = pltpu.async_copy(pe_hbm.at[pl.ds(lw, l_per_w)], pe_v, psem)
        idescs = [
            pltpu.async_copy(
                src_hbm.at[b, pl.ds(lw, l_per_w)], idx_v.at[b], isem
            )
            for b in range(batch)
        ]
        for d in idescs:
            d.wait()

        def start_gather(c):
            b, h = divmod(c, 2)
            return pltpu.async_copy(
                table_hbm.at[idx_v.at[b, pl.ds(h * half, half)]],
                rows[c % NBUF],
                gsem[c % NBUF],
            )

        def start_store(c):
            b, h = divmod(c, 2)
            return pltpu.async_copy(
                rows[c % NBUF],
                out_hbm.at[pl.ds(b * seq_len + lw + h * half, half)],
                ssem[c % NBUF],
            )

        gdescs, sdescs = {}, {}
        for c in range(NBUF):
            gdescs[c] = start_gather(c)
        pe_desc.wait()

        for c in range(n_chunks):
            if c >= 2 and c + 1 < n_chunks:
                sdescs[c - 2].wait()
                gdescs[c + 1] = start_gather(c + 1)
            gdescs[c].wait()

            h = c % 2
            rbuf = rows[c % NBUF]

            @plsc.parallel_loop(0, half, step=1, unroll=4)
            def _(r, rbuf=rbuf, h=h):
                for j in range(n_slices):
                    sl = pl.ds(j * LANES, LANES)
                    rbuf[r, sl] = rbuf[r, sl] * scale + pe_v[h * half + r, sl]

            sdescs[c] = start_store(c)

        for c in range(n_chunks - NBUF, n_chunks):
            sdescs[c].wait()

    return k


def kernel(src, tok_embedding, pe):
    batch, seq_len = src.shape
    d_model = tok_embedding.shape[1]
    src2d = src.astype(jnp.int32)
    pe2d = pe[0, :seq_len, :]
    k = _make_sc_kernel(batch, seq_len, d_model)
    out = k(src2d, tok_embedding, pe2d)
    return out.reshape(batch, seq_len, d_model)


# speedup vs baseline: 1.3015x; 1.3015x over previous
import functools
import math

import jax
import jax.numpy as jnp
from jax import lax
from jax.experimental import pallas as pl
from jax.experimental.pallas import tpu as pltpu
from jax.experimental.pallas import tpu_sc as plsc

LANES = 16
NBUF = 4
GRAN = 8


@functools.lru_cache(maxsize=None)
def _make_sc_kernel(batch: int, seq_len: int, d_model: int):
    info = plsc.get_sparse_core_info()
    num_workers = info.num_cores * info.num_subcores
    l_per_w = seq_len // num_workers
    n_sc = l_per_w // GRAN
    n_slices = d_model // LANES
    scale = math.sqrt(float(d_model))
    mesh = plsc.VectorSubcoreMesh(core_axis_name="c", subcore_axis_name="s")

    @functools.partial(
        pl.kernel,
        mesh=mesh,
        out_type=jax.ShapeDtypeStruct((batch * seq_len, d_model), jnp.float32),
        scratch_types=[
            pltpu.VMEM((batch, l_per_w), jnp.int32),
        ]
        + [pltpu.VMEM((batch * GRAN, d_model), jnp.float32) for _ in range(NBUF)]
        + [pltpu.VMEM((GRAN, d_model), jnp.float32) for _ in range(2)]
        + [pltpu.SemaphoreType.DMA for _ in range(3 + 2 * NBUF)],
    )
    def k(src_hbm, table_hbm, pe_hbm, out_hbm, idx_v, r0, r1, r2, r3,
          pe0, pe1, isem, p0, p1, g0, g1, g2, g3, s0, s1, s2, s3):
        rows = [r0, r1, r2, r3]
        pes = [pe0, pe1]
        psem = [p0, p1]
        gsem = [g0, g1, g2, g3]
        ssem = [s0, s1, s2, s3]
        wid = lax.axis_index("s") * info.num_cores + lax.axis_index("c")
        lw = wid * l_per_w

        def start_pe(sc):
            return pltpu.async_copy(
                pe_hbm.at[pl.ds(lw + sc * GRAN, GRAN)], pes[sc % 2], psem[sc % 2]
            )

        def start_gathers(sc):
            p = sc % NBUF
            return [
                pltpu.async_copy(
                    table_hbm.at[idx_v.at[b, pl.ds(sc * GRAN, GRAN)]],
                    rows[p].at[pl.ds(b * GRAN, GRAN)],
                    gsem[p],
                )
                for b in range(batch)
            ]

        def start_stores(sc):
            p = sc % NBUF
            return [
                pltpu.async_copy(
                    rows[p].at[pl.ds(b * GRAN, GRAN)],
                    out_hbm.at[pl.ds(b * seq_len + lw + sc * GRAN, GRAN)],
                    ssem[p],
                )
                for b in range(batch)
            ]

        pdescs = {0: start_pe(0), 1: start_pe(1)}
        idescs = [
            pltpu.async_copy(
                src_hbm.at[b, pl.ds(lw, l_per_w)], idx_v.at[b], isem
            )
            for b in range(batch)
        ]
        for d in idescs:
            d.wait()

        gdescs, sdescs = {}, {}
        for sc in range(NBUF):
            gdescs[sc] = start_gathers(sc)

        for sc in range(n_sc):
            if sc >= NBUF - 1 and sc + 1 < n_sc:
                for d in sdescs[sc + 1 - NBUF]:
                    d.wait()
                gdescs[sc + 1] = start_gathers(sc + 1)
            for d in gdescs[sc]:
                d.wait()
            pdescs[sc].wait()

            rbuf = rows[sc % NBUF]
            pv = pes[sc % 2]

            @plsc.parallel_loop(0, GRAN * n_slices, step=1, unroll=2)
            def _(t, rbuf=rbuf, pv=pv):
                r = lax.rem(t, GRAN)
                j = lax.div(t, GRAN)
                sl = pl.ds(j * LANES, LANES)
                pvec = pv[r, sl]
                for b in range(batch):
                    row = b * GRAN + r
                    rbuf[row, sl] = rbuf[row, sl] * scale + pvec

            if sc + 2 < n_sc:
                pdescs[sc + 2] = start_pe(sc + 2)
            sdescs[sc] = start_stores(sc)

        for sc in range(n_sc - NBUF, n_sc):
            for d in sdescs[sc]:
                d.wait()

    return k


def kernel(src, tok_embedding, pe):
    batch, seq_len = src.shape
    d_model = tok_embedding.shape[1]
    src2d = src.astype(jnp.int32)
    pe2d = pe[0, :seq_len, :]
    k = _make_sc_kernel(batch, seq_len, d_model)
    out = k(src2d, tok_embedding, pe2d)
    return out.reshape(batch, seq_len, d_model)
